# Initial kernel scaffold; baseline (speedup 1.0000x reference)
import jax
import jax.numpy as jnp
from jax.experimental import pallas as pl


def kernel(x, neighbor, W1, b1, W2, b2, Wc, bc):
    raise NotImplementedError("write your pallas kernel here")


# speedup vs baseline: 1.8183x; 1.8183x over previous
import jax
import jax.numpy as jnp
from jax.experimental import pallas as pl
from jax.experimental.pallas import tpu as pltpu

N = 10000
K = 32
FEAT = 128
H1, H2 = 64, 32
NUM_CLASS = 40
ALPHA = 0.1
BN = 200

_EPS = 1e-12


def _body(x_ref, nb_ref, w1_ref, b1_ref, w2_ref, b2_ref, wc_ref, bc_ref,
          out_ref):
    f32 = jnp.float32
    one_m_a = f32(1.0 - ALPHA)

    nb = nb_ref[...]
    inv = jax.lax.rsqrt(jnp.maximum(jnp.sum(nb * nb, axis=1, keepdims=True),
                                    _EPS * _EPS))
    nb = nb * inv
    nbh = (jnp.dot(nb, w1_ref[...], preferred_element_type=f32)
           + b1_ref[...])
    nbh3 = nbh.reshape(BN, K, H1)
    s1 = jnp.sum(nbh3, axis=1)
    r1 = jnp.sum(jnp.maximum(nbh3, 0.0), axis=1)
    nb2 = (jnp.dot(nbh, w2_ref[...], preferred_element_type=f32)
           + b2_ref[...])
    nb23 = nb2.reshape(BN, K, H2)
    s2 = jnp.sum(nb23, axis=1)
    r2 = jnp.sum(jnp.maximum(nb23, 0.0), axis=1)

    xb = x_ref[...]
    xinv = jax.lax.rsqrt(jnp.maximum(jnp.sum(xb * xb, axis=1, keepdims=True),
                                     _EPS * _EPS))
    xn = xb * xinv
    h = jnp.dot(xn, w1_ref[...], preferred_element_type=f32) + b1_ref[...]
    x1 = jnp.maximum(h + one_m_a * s1, 0.0)
    x2 = one_m_a * (x1 + r1) + f32(ALPHA) * h
    h2 = jnp.dot(x2, w2_ref[...], preferred_element_type=f32) + b2_ref[...]
    x3 = jnp.maximum(h2 + one_m_a * s2, 0.0)
    x4 = one_m_a * (x3 + r2) + f32(ALPHA) * h2
    out_ref[...] = (jnp.dot(x4, wc_ref[...], preferred_element_type=f32)
                    + bc_ref[...])


def kernel(x, neighbor, W1, b1, W2, b2, Wc, bc):
    nb_flat = neighbor.reshape(N * K, FEAT)
    w1t = W1.T
    w2t = W2.T
    wct = Wc.T
    b1r = b1.reshape(1, H1)
    b2r = b2.reshape(1, H2)
    bcr = bc.reshape(1, NUM_CLASS)

    grid = (N // BN,)
    rep = lambda i: (0, 0)
    out = pl.pallas_call(
        _body,
        grid=grid,
        in_specs=[
            pl.BlockSpec((BN, FEAT), lambda i: (i, 0)),
            pl.BlockSpec((BN * K, FEAT), lambda i: (i, 0)),
            pl.BlockSpec((FEAT, H1), rep),
            pl.BlockSpec((1, H1), rep),
            pl.BlockSpec((H1, H2), rep),
            pl.BlockSpec((1, H2), rep),
            pl.BlockSpec((H2, NUM_CLASS), rep),
            pl.BlockSpec((1, NUM_CLASS), rep),
        ],
        out_specs=pl.BlockSpec((BN, NUM_CLASS), lambda i: (i, 0)),
        out_shape=jax.ShapeDtypeStruct((N, NUM_CLASS), jnp.float32),
        compiler_params=pltpu.CompilerParams(
            dimension_semantics=("arbitrary",)),
    )(x, nb_flat, w1t, b1r, w2t, b2r, wct, bcr)
    return out
